# baseline (speedup 1.0000x reference)
import functools
import math

import jax
import jax.numpy as jnp
from jax import lax
from jax.experimental import pallas as pl
from jax.experimental.pallas import tpu as pltpu
from jax.experimental.pallas import tpu_sc as plsc

_NUM_DOSES = 100
_B = 16384
_D = 64
_NC = 2
_NS = 16
_NW = _NC * _NS
_BPW = _B // _NW
_NCHUNK = _BPW // 128
_L = 16


def _sc_gather(dev3, dose3, table):
    mesh = plsc.VectorSubcoreMesh(core_axis_name="c", subcore_axis_name="s")

    @functools.partial(
        pl.kernel,
        mesh=mesh,
        out_type=jax.ShapeDtypeStruct((_B, _D), jnp.float32),
        scratch_types=[
            pltpu.VMEM((_NCHUNK, 128), jnp.int32),
            pltpu.VMEM((_NCHUNK, 128), jnp.int32),
            pltpu.VMEM((_BPW, _D), jnp.float32),
            pltpu.SemaphoreType.DMA,
        ],
        compiler_params=pltpu.CompilerParams(use_tc_tiling_on_sc=False),
    )
    def k(dev_hbm, dose_hbm, table_hbm, out_hbm, dev_v, dose_v, rows_v, sem):
        wid = lax.axis_index("s") * _NC + lax.axis_index("c")
        base = wid * _BPW
        pltpu.sync_copy(dev_hbm.at[wid], dev_v)
        pltpu.sync_copy(dose_hbm.at[wid], dose_v)
        for j in range(_NCHUNK):
            for i in range(128 // _L):
                sl = pl.ds(i * _L, _L)
                dev_v[j, sl] = dev_v[j, sl] * _NUM_DOSES + dose_v[j, sl]
        handles = []
        for j in range(_NCHUNK):
            handles.append(
                pltpu.async_copy(
                    table_hbm.at[dev_v.at[j]],
                    rows_v.at[pl.ds(j * 128, 128)],
                    sem,
                )
            )
        for h in handles:
            h.wait()
        pltpu.sync_copy(rows_v, out_hbm.at[pl.ds(base, _BPW)])

    return k(dev3, dose3, table)


_BLK = 2048


def _mlp_body(emb_ref, w1_ref, b1_ref, w2_ref, b2_ref, out_ref):
    emb = emb_ref[...]
    h = jnp.dot(emb, w1_ref[...], preferred_element_type=jnp.float32)
    h = h + b1_ref[...]
    h = 0.5 * h * (1.0 + lax.erf(h * (1.0 / math.sqrt(2.0))))
    o = jnp.dot(h, w2_ref[...], preferred_element_type=jnp.float32)
    out_ref[...] = o + b2_ref[...] + emb


def _mlp(emb, W1, b1, W2, b2):
    grid = (_B // _BLK,)
    return pl.pallas_call(
        _mlp_body,
        grid=grid,
        in_specs=[
            pl.BlockSpec((_BLK, _D), lambda i: (i, 0)),
            pl.BlockSpec((_D, 2 * _D), lambda i: (0, 0)),
            pl.BlockSpec((1, 2 * _D), lambda i: (0, 0)),
            pl.BlockSpec((2 * _D, _D), lambda i: (0, 0)),
            pl.BlockSpec((1, _D), lambda i: (0, 0)),
        ],
        out_specs=pl.BlockSpec((_BLK, _D), lambda i: (i, 0)),
        out_shape=jax.ShapeDtypeStruct((_B, _D), jnp.float32),
    )(emb, W1, b1, W2, b2)


def kernel(table, W1, b1, W2, b2, device_idx, dose_idx):
    dev3 = device_idx.astype(jnp.int32).reshape(_NW, _NCHUNK, 128)
    dose3 = dose_idx.astype(jnp.int32).reshape(_NW, _NCHUNK, 128)
    emb = _sc_gather(dev3, dose3, table)
    return _mlp(emb, W1, b1.reshape(1, -1), W2, b2.reshape(1, -1))


# speedup vs baseline: 2.1702x; 2.1702x over previous
import functools
import math

import jax
import jax.numpy as jnp
from jax import lax
from jax.experimental import pallas as pl
from jax.experimental.pallas import tpu as pltpu
from jax.experimental.pallas import tpu_sc as plsc

_NUM_DOSES = 100
_B = 16384
_D = 64
_NC = 2
_NS = 16
_NW = _NC * _NS
_BPW = _B // _NW
_CHUNK = 64
_NCHUNK = _BPW // _CHUNK
_L = 16


def _sc_gather(dev, dose, table3):
    mesh = plsc.VectorSubcoreMesh(core_axis_name="c", subcore_axis_name="s")

    @functools.partial(
        pl.kernel,
        mesh=mesh,
        out_type=jax.ShapeDtypeStruct((_B, _D), jnp.float32),
        scratch_types=[
            pltpu.VMEM((_BPW + _L,), jnp.int32),
            pltpu.VMEM((_BPW,), jnp.int32),
            pltpu.VMEM((_CHUNK, 8, _D), jnp.float32),
            pltpu.VMEM((_CHUNK, _D), jnp.float32),
            pltpu.SemaphoreType.DMA,
        ],
        compiler_params=pltpu.CompilerParams(use_tc_tiling_on_sc=True),
    )
    def k(dev_hbm, dose_hbm, table_hbm, out_hbm,
          idx_v, dose_v, buf_v, rows_v, sem):
        wid = lax.axis_index("s") * _NC + lax.axis_index("c")
        base = wid * _BPW
        pltpu.sync_copy(dev_hbm.at[pl.ds(base, _BPW)], idx_v.at[pl.ds(0, _BPW)])
        pltpu.sync_copy(dose_hbm.at[pl.ds(base, _BPW)], dose_v)
        for i in range(_BPW // _L):
            sl = pl.ds(i * _L, _L)
            idx_v[sl] = idx_v[sl] * _NUM_DOSES + dose_v[sl]

        for j in range(_NCHUNK):
            def fire(kk, carry):
                c = idx_v[pl.ds(j * _CHUNK + kk, _L)][0]
                t = lax.shift_right_logical(c, 3)
                pltpu.async_copy(table_hbm.at[t], buf_v.at[kk], sem)
                return carry

            lax.fori_loop(0, _CHUNK, fire, 0)

            def drain(kk, carry):
                pltpu.make_async_copy(table_hbm.at[0], buf_v.at[kk], sem).wait()
                return carry

            lax.fori_loop(0, _CHUNK, drain, 0)

            def select(kk, carry):
                r = lax.bitwise_and(idx_v[pl.ds(j * _CHUNK + kk, _L)][0], 7)
                for m in range(_D // _L):
                    sl = pl.ds(m * _L, _L)
                    rows_v[kk, sl] = buf_v[kk, r, sl]
                return carry

            lax.fori_loop(0, _CHUNK, select, 0)
            pltpu.sync_copy(rows_v, out_hbm.at[pl.ds(base + j * _CHUNK, _CHUNK)])

    return k(dev, dose, table3)


_BLK = 2048


def _mlp_body(emb_ref, w1_ref, b1_ref, w2_ref, b2_ref, out_ref):
    emb = emb_ref[...]
    h = jnp.dot(emb, w1_ref[...], preferred_element_type=jnp.float32)
    h = h + b1_ref[...]
    h = 0.5 * h * (1.0 + lax.erf(h * (1.0 / math.sqrt(2.0))))
    o = jnp.dot(h, w2_ref[...], preferred_element_type=jnp.float32)
    out_ref[...] = o + b2_ref[...] + emb


def _mlp(emb, W1, b1, W2, b2):
    grid = (_B // _BLK,)
    return pl.pallas_call(
        _mlp_body,
        grid=grid,
        in_specs=[
            pl.BlockSpec((_BLK, _D), lambda i: (i, 0)),
            pl.BlockSpec((_D, 2 * _D), lambda i: (0, 0)),
            pl.BlockSpec((1, 2 * _D), lambda i: (0, 0)),
            pl.BlockSpec((2 * _D, _D), lambda i: (0, 0)),
            pl.BlockSpec((1, _D), lambda i: (0, 0)),
        ],
        out_specs=pl.BlockSpec((_BLK, _D), lambda i: (i, 0)),
        out_shape=jax.ShapeDtypeStruct((_B, _D), jnp.float32),
    )(emb, W1, b1, W2, b2)


def kernel(table, W1, b1, W2, b2, device_idx, dose_idx):
    dev = device_idx.astype(jnp.int32)
    dose = dose_idx.astype(jnp.int32)
    table3 = table.reshape(125000, 8, _D)
    emb = _sc_gather(dev, dose, table3)
    return _mlp(emb, W1, b1.reshape(1, -1), W2, b2.reshape(1, -1))
